# Initial kernel scaffold; baseline (speedup 1.0000x reference)
import jax
import jax.numpy as jnp
from jax.experimental import pallas as pl


def kernel(features, coords, edge_index, edge_attr, edge_sh, c_noise, W_attr, W_sh, W_out, w_scale):
    raise NotImplementedError("write your pallas kernel here")


# speedup vs baseline: 78.4562x; 78.4562x over previous
import functools
import math

import jax
import jax.numpy as jnp
from jax import lax
from jax.experimental import pallas as pl
from jax.experimental.pallas import tpu as pltpu
from jax.experimental.pallas import tpu_sc as plsc

_N = 10000
_E = 320000
_D = 128
_SH = 9
_EA = 16
_INV_SQRT_DEG = 1.0 / math.sqrt(32.0)

_NC = 2
_NS = 16
_NW = _NC * _NS
_EPW = _E // _NW
_G = 80
_NCHUNK = _EPW // _G
_NP = 10240
_RPS = _NP // _NS
_ZR = 128

_BE = 2000
_BN = 2000


def _i0(i):
    return i * 0


def _coef_body(ea_ref, sh_ref, wa_ref, ws_ref, o_ref):
    gate_lin = lax.dot_general(
        ea_ref[...], wa_ref[...], (((1,), (0,)), ((), ())),
        precision=lax.Precision.HIGHEST, preferred_element_type=jnp.float32)
    sh_mod = lax.dot_general(
        sh_ref[...], ws_ref[...], (((1,), (0,)), ((), ())),
        precision=lax.Precision.HIGHEST, preferred_element_type=jnp.float32)
    o_ref[...] = jax.nn.sigmoid(gate_lin) * sh_mod


def _compute_coef(edge_attr, edge_sh, W_attr, W_sh):
    return pl.pallas_call(
        _coef_body,
        grid=(_E // _BE,),
        in_specs=[
            pl.BlockSpec((_BE, _EA), lambda i: (i, _i0(i))),
            pl.BlockSpec((_BE, _SH), lambda i: (i, _i0(i))),
            pl.BlockSpec((_EA, _D), lambda i: (_i0(i), _i0(i))),
            pl.BlockSpec((_SH, _D), lambda i: (_i0(i), _i0(i))),
        ],
        out_specs=pl.BlockSpec((_BE, _D), lambda i: (i, _i0(i))),
        out_shape=jax.ShapeDtypeStruct((_E, _D), jnp.float32),
    )(edge_attr, edge_sh, W_attr, W_sh)


def _loop32(lo, hi, body_fn):
    lax.fori_loop(jnp.int32(lo), jnp.int32(hi),
                  lambda i, c: (body_fn(i), c)[1], None)


def _sc_body(feat_hbm, src_hbm, dst_hbm, coef_hbm, out_hbm,
             sidx, didx, rows, coefb, stage, agg_sh, sem):
    cid = lax.axis_index("c").astype(jnp.int32)
    sid = lax.axis_index("s").astype(jnp.int32)
    wid = cid * jnp.int32(_NS) + sid

    def _zero_stage(i):
        def _zs(j16):
            stage.at[i, pl.ds(j16 * jnp.int32(16), 16)][...] = (
                jnp.zeros((16,), jnp.float32))
        _loop32(0, _D // 16, _zs)
    _loop32(0, _ZR, _zero_stage)

    for r in range(0, _RPS, _ZR):
        pltpu.sync_copy(stage, agg_sh.at[pl.ds(sid * jnp.int32(_RPS) + jnp.int32(r), _ZR)])
    plsc.subcore_barrier()

    def _chunk(k):
        off = wid * jnp.int32(_EPW) + k * jnp.int32(_G)
        pltpu.sync_copy(src_hbm.at[pl.ds(off, _G)], sidx)
        pltpu.sync_copy(dst_hbm.at[pl.ds(off, _G)], didx)
        pltpu.async_copy(feat_hbm.at[sidx], rows, sem).wait()
        pltpu.sync_copy(coef_hbm.at[pl.ds(off, _G)], coefb)

        def _row(i):
            def _mul(j16):
                j = j16 * jnp.int32(16)
                rows.at[i, pl.ds(j, 16)][...] = (
                    rows.at[i, pl.ds(j, 16)][...]
                    * coefb.at[i, pl.ds(j, 16)][...])
            _loop32(0, _D // 16, _mul)
        _loop32(0, _G, _row)

        pltpu.sync_copy(rows, agg_sh.at[didx], add=True)
    _loop32(0, _NCHUNK, _chunk)

    plsc.subcore_barrier()

    for r in range(0, _RPS, _ZR):
        row0 = sid * jnp.int32(_RPS) + jnp.int32(r)
        pltpu.sync_copy(agg_sh.at[pl.ds(row0, _ZR)], stage)
        pltpu.sync_copy(stage, out_hbm.at[cid, pl.ds(row0, _ZR)])


def _sc_scatter(features, src, dst, coef):
    mesh = plsc.VectorSubcoreMesh(core_axis_name="c", subcore_axis_name="s")
    k = pl.kernel(
        _sc_body,
        mesh=mesh,
        out_type=jax.ShapeDtypeStruct((_NC, _NP, _D), jnp.float32),
        scratch_types=[
            pltpu.VMEM((_G,), jnp.int32),
            pltpu.VMEM((_G,), jnp.int32),
            pltpu.VMEM((_G, _D), jnp.float32),
            pltpu.VMEM((_G, _D), jnp.float32),
            pltpu.VMEM((_ZR, _D), jnp.float32),
            pltpu.VMEM_SHARED((_NP, _D), jnp.float32),
            pltpu.SemaphoreType.DMA,
        ],
    )
    return k(features, src, dst, coef)


def _out_body(agg_ref, wout_ref, cn_ref, ws_ref, o_ref):
    agg = (agg_ref[0] + agg_ref[1]) * _INV_SQRT_DEG
    out = lax.dot_general(
        agg, wout_ref[...], (((1,), (0,)), ((), ())),
        precision=lax.Precision.HIGHEST, preferred_element_type=jnp.float32)
    scale = 1.0 + jnp.tanh(cn_ref[...]) * ws_ref[...]
    o_ref[...] = out * scale


def _epilogue(partials, W_out, c_noise, w_scale):
    return pl.pallas_call(
        _out_body,
        grid=(_N // _BN,),
        in_specs=[
            pl.BlockSpec((_NC, _BN, _D), lambda i: (_i0(i), i, _i0(i))),
            pl.BlockSpec((_D, _D), lambda i: (_i0(i), _i0(i))),
            pl.BlockSpec((_BN, 1), lambda i: (i, _i0(i))),
            pl.BlockSpec((1, _D), lambda i: (_i0(i), _i0(i))),
        ],
        out_specs=pl.BlockSpec((_BN, _D), lambda i: (i, _i0(i))),
        out_shape=jax.ShapeDtypeStruct((_N, _D), jnp.float32),
    )(partials, W_out, c_noise, w_scale)


def kernel(features, coords, edge_index, edge_attr, edge_sh, c_noise,
           W_attr, W_sh, W_out, w_scale):
    out_dtype = jnp.result_type(features.dtype, W_attr.dtype, W_out.dtype)
    src = edge_index[0].astype(jnp.int32)
    dst = edge_index[1].astype(jnp.int32)
    coef = _compute_coef(edge_attr, edge_sh,
                         W_attr.astype(jnp.float32), W_sh.astype(jnp.float32))
    partials = _sc_scatter(features, src, dst, coef)
    new_features = _epilogue(
        partials, W_out.astype(jnp.float32),
        c_noise.reshape(_N, 1).astype(jnp.float32),
        w_scale.reshape(1, _D).astype(jnp.float32))
    return (coords, new_features.astype(out_dtype))


# speedup vs baseline: 87.2935x; 1.1126x over previous
import functools
import math

import jax
import jax.numpy as jnp
from jax import lax
from jax.experimental import pallas as pl
from jax.experimental.pallas import tpu as pltpu
from jax.experimental.pallas import tpu_sc as plsc

_N = 10000
_E = 320000
_D = 128
_SH = 9
_EA = 16
_INV_SQRT_DEG = 1.0 / math.sqrt(32.0)

_NC = 2
_NS = 16
_NW = _NC * _NS
_EPW = _E // _NW
_G = 80
_NCHUNK = _EPW // _G
_NP = 10240
_RPS = _NP // _NS
_ZR = 128

_BE = 2000
_BN = 2000


def _i0(i):
    return i * 0


def _coef_body(ea_ref, sh_ref, w_ref, o_ref):
    x = jnp.concatenate(
        [ea_ref[...], sh_ref[...],
         jnp.zeros((_BE, 32 - _EA - _SH), jnp.float32)], axis=1)
    y = lax.dot_general(
        x, w_ref[...], (((1,), (0,)), ((), ())),
        preferred_element_type=jnp.float32)
    o_ref[...] = jax.nn.sigmoid(y[:, :_D]) * y[:, _D:]


def _compute_coef(edge_attr, edge_sh, W_cat):
    return pl.pallas_call(
        _coef_body,
        grid=(_E // _BE,),
        in_specs=[
            pl.BlockSpec((_BE, _EA), lambda i: (i, _i0(i))),
            pl.BlockSpec((_BE, _SH), lambda i: (i, _i0(i))),
            pl.BlockSpec((32, 2 * _D), lambda i: (_i0(i), _i0(i))),
        ],
        out_specs=pl.BlockSpec((_BE, _D), lambda i: (i, _i0(i))),
        out_shape=jax.ShapeDtypeStruct((_E, _D), jnp.float32),
    )(edge_attr, edge_sh, W_cat)


def _loop32(lo, hi, body_fn):
    lax.fori_loop(jnp.int32(lo), jnp.int32(hi),
                  lambda i, c: (body_fn(i), c)[1], None)


def _sc_body(feat_hbm, eidx_hbm, coef_hbm, out_hbm,
             sidx, didx, rows, coefb, stage, agg_sh, sem):
    cid = lax.axis_index("c").astype(jnp.int32)
    sid = lax.axis_index("s").astype(jnp.int32)
    wid = cid * jnp.int32(_NS) + sid

    def _zero_stage(i):
        def _zs(j16):
            stage.at[i, pl.ds(j16 * jnp.int32(16), 16)][...] = (
                jnp.zeros((16,), jnp.float32))
        _loop32(0, _D // 16, _zs)
    _loop32(0, _ZR, _zero_stage)

    for r in range(0, _RPS, _ZR):
        pltpu.sync_copy(stage, agg_sh.at[pl.ds(sid * jnp.int32(_RPS) + jnp.int32(r), _ZR)])
    plsc.subcore_barrier()

    def _chunk(k):
        off = wid * jnp.int32(_EPW) + k * jnp.int32(_G)
        pltpu.sync_copy(eidx_hbm.at[pl.ds(off, _G)], sidx)
        pltpu.sync_copy(eidx_hbm.at[pl.ds(off + jnp.int32(_E), _G)], didx)
        pltpu.async_copy(feat_hbm.at[sidx], rows, sem).wait()
        pltpu.sync_copy(coef_hbm.at[pl.ds(off, _G)], coefb)

        def _row(i):
            def _mul(j16):
                j = j16 * jnp.int32(16)
                rows.at[i, pl.ds(j, 16)][...] = (
                    rows.at[i, pl.ds(j, 16)][...]
                    * coefb.at[i, pl.ds(j, 16)][...])
            _loop32(0, _D // 16, _mul)
        _loop32(0, _G, _row)

        pltpu.sync_copy(rows, agg_sh.at[didx], add=True)
    _loop32(0, _NCHUNK, _chunk)

    plsc.subcore_barrier()

    for r in range(0, _RPS, _ZR):
        row0 = sid * jnp.int32(_RPS) + jnp.int32(r)
        pltpu.sync_copy(agg_sh.at[pl.ds(row0, _ZR)], stage)
        pltpu.sync_copy(stage, out_hbm.at[cid, pl.ds(row0, _ZR)])


def _sc_scatter(features, eidx, coef):
    mesh = plsc.VectorSubcoreMesh(core_axis_name="c", subcore_axis_name="s")
    k = pl.kernel(
        _sc_body,
        mesh=mesh,
        out_type=jax.ShapeDtypeStruct((_NC, _NP, _D), jnp.float32),
        scratch_types=[
            pltpu.VMEM((_G,), jnp.int32),
            pltpu.VMEM((_G,), jnp.int32),
            pltpu.VMEM((_G, _D), jnp.float32),
            pltpu.VMEM((_G, _D), jnp.float32),
            pltpu.VMEM((_ZR, _D), jnp.float32),
            pltpu.VMEM_SHARED((_NP, _D), jnp.float32),
            pltpu.SemaphoreType.DMA,
        ],
    )
    return k(features, eidx, coef)


def _out_body(agg_ref, wout_ref, cn_ref, ws_ref, o_ref):
    agg = (agg_ref[0] + agg_ref[1]) * _INV_SQRT_DEG
    out = lax.dot_general(
        agg, wout_ref[...], (((1,), (0,)), ((), ())),
        precision=lax.Precision.HIGHEST, preferred_element_type=jnp.float32)
    scale = 1.0 + jnp.tanh(cn_ref[...]) * ws_ref[...]
    o_ref[...] = out * scale


def _epilogue(partials, W_out, c_noise, w_scale):
    return pl.pallas_call(
        _out_body,
        grid=(_N // _BN,),
        in_specs=[
            pl.BlockSpec((_NC, _BN, _D), lambda i: (_i0(i), i, _i0(i))),
            pl.BlockSpec((_D, _D), lambda i: (_i0(i), _i0(i))),
            pl.BlockSpec((_BN, 1), lambda i: (i, _i0(i))),
            pl.BlockSpec((1, _D), lambda i: (_i0(i), _i0(i))),
        ],
        out_specs=pl.BlockSpec((_BN, _D), lambda i: (i, _i0(i))),
        out_shape=jax.ShapeDtypeStruct((_N, _D), jnp.float32),
    )(partials, W_out, c_noise, w_scale)


def kernel(features, coords, edge_index, edge_attr, edge_sh, c_noise,
           W_attr, W_sh, W_out, w_scale):
    out_dtype = jnp.result_type(features.dtype, W_attr.dtype, W_out.dtype)
    eidx = edge_index.reshape(-1).astype(jnp.int32)
    W_cat = jnp.zeros((32, 2 * _D), jnp.float32)
    W_cat = W_cat.at[:_EA, :_D].set(W_attr.astype(jnp.float32))
    W_cat = W_cat.at[_EA:_EA + _SH, _D:].set(W_sh.astype(jnp.float32))
    coef = _compute_coef(edge_attr, edge_sh, W_cat)
    partials = _sc_scatter(features, eidx, coef)
    new_features = _epilogue(
        partials, W_out.astype(jnp.float32),
        c_noise.reshape(_N, 1).astype(jnp.float32),
        w_scale.reshape(1, _D).astype(jnp.float32))
    return (coords, new_features.astype(out_dtype))
